# baseline (speedup 1.0000x reference)
import functools

import jax
import jax.numpy as jnp
from jax import lax
from jax.experimental import pallas as pl
from jax.experimental.pallas import tpu as pltpu
from jax.experimental.pallas import tpu_sc as plsc

N = 10000
E = 160000
D = 256
HALF = 128
NS = 16
CHUNK = 128
NBUF = 1
CHUNKS_STORED = 80
CHUNKS_PER_TILE = 79
EPT = CHUNKS_STORED * CHUNK
PAD_PER_TILE = EPT - E // NS
DUMMY_PER_TILE = 8
N_PAD = N + NS * DUMMY_PER_TILE
ROWS_PER_TILE = 624
TAIL_ROWS = N - NS * ROWS_PER_TILE
BN_EPS = 1e-5


def _sc_agg_body(t_lo, t_hi, i_lo, i_hi, src_r, dst_r,
                 out_lo, out_hi, acc, src_v, dst_v,
                 buf0, buf1, sg0, sg1, ss0, ss1):
    c = lax.axis_index("c")
    s = lax.axis_index("s")
    r0 = pl.multiple_of(s * ROWS_PER_TILE, 8)
    tail0 = NS * ROWS_PER_TILE

    def run_half(tab, ini, out):
        pltpu.sync_copy(ini.at[pl.ds(r0, ROWS_PER_TILE)],
                        acc.at[pl.ds(r0, ROWS_PER_TILE)])

        @pl.when(s == NS - 1)
        def _():
            pltpu.sync_copy(ini.at[pl.ds(tail0, TAIL_ROWS)],
                            acc.at[pl.ds(tail0, TAIL_ROWS)])

        plsc.subcore_barrier()

        def pair(t0, t1):
            d0 = pltpu.async_copy(tab.at[src_v.at[t0]], buf0, sg0)
            d1 = pltpu.async_copy(tab.at[src_v.at[t1]], buf1, sg1)
            d0.wait()
            s0 = pltpu.async_copy(buf0, acc.at[dst_v.at[t0]], ss0,
                                  add=True)
            d1.wait()
            s1 = pltpu.async_copy(buf1, acc.at[dst_v.at[t1]], ss1,
                                  add=True)
            s0.wait()
            s1.wait()

        for p0, pn, do in ((0, 64, 64), (64, 16, 15)):
            pltpu.sync_copy(src_r.at[s, pl.ds(p0, pn)],
                            src_v.at[pl.ds(0, pn)])
            pltpu.sync_copy(dst_r.at[s, pl.ds(p0, pn)],
                            dst_v.at[pl.ds(0, pn)])

            def body(g, carry):
                pair(g * 2, g * 2 + 1)
                return carry

            lax.fori_loop(0, do // 2, body, 0)
            if do % 2:
                tl = do - 1
                dl = pltpu.async_copy(tab.at[src_v.at[tl]], buf0, sg0)
                dl.wait()
                sl = pltpu.async_copy(buf0, acc.at[dst_v.at[tl]], ss0,
                                      add=True)
                sl.wait()
        plsc.subcore_barrier()
        pltpu.sync_copy(acc.at[pl.ds(r0, ROWS_PER_TILE)],
                        out.at[pl.ds(r0, ROWS_PER_TILE)])

        @pl.when(s == NS - 1)
        def _():
            pltpu.sync_copy(acc.at[pl.ds(tail0, TAIL_ROWS)],
                            out.at[pl.ds(tail0, TAIL_ROWS)])

    pl.when(c == 0)(lambda: run_half(t_lo, i_lo, out_lo))
    pl.when(c == 1)(lambda: run_half(t_hi, i_hi, out_hi))


def _sc_agg(table_lo, table_hi, init_lo, init_hi, src_r, dst_r):
    mesh = plsc.VectorSubcoreMesh(core_axis_name="c", subcore_axis_name="s")
    f = pl.kernel(
        _sc_agg_body,
        out_type=(
            jax.ShapeDtypeStruct((N, HALF), jnp.float32),
            jax.ShapeDtypeStruct((N, HALF), jnp.float32),
        ),
        mesh=mesh,
        scratch_types=(
            [
                pltpu.VMEM_SHARED((N_PAD, HALF), jnp.float32),
                pltpu.VMEM((64, CHUNK), jnp.int32),
                pltpu.VMEM((64, CHUNK), jnp.int32),
            ]
            + [pltpu.VMEM((CHUNK, HALF), jnp.float32)] * 2
            + [pltpu.SemaphoreType.DMA] * 4
        ),
    )
    return f(table_lo, table_hi, init_lo, init_hi, src_r, dst_r)


ROW_BLK = 5000
GRID = N // ROW_BLK


def _prep_body(x_ref, xlo_ref, xhi_ref, rlo_ref, rhi_ref):
    x = x_ref[...]
    xlo_ref[...] = x[:, :HALF]
    xhi_ref[...] = x[:, HALF:]
    r = jnp.maximum(x, 0.0)
    rlo_ref[...] = r[:, :HALF]
    rhi_ref[...] = r[:, HALF:]


def _prep(x):
    return pl.pallas_call(
        _prep_body,
        grid=(GRID,),
        in_specs=[pl.BlockSpec((ROW_BLK, D), lambda i: (i, 0))],
        out_specs=tuple(pl.BlockSpec((ROW_BLK, HALF), lambda i: (i, 0))
                        for _ in range(4)),
        out_shape=tuple(jax.ShapeDtypeStruct((N, HALF), jnp.float32)
                        for _ in range(4)),
    )(x)


def _m1_body(alo_ref, ahi_ref, xlo_ref, xhi_ref, eps_ref, w_ref, b_ref,
             h_ref, s_ref, ss_ref):
    eps = eps_ref[0]
    a = jnp.concatenate(
        [alo_ref[...] + eps * xlo_ref[...],
         ahi_ref[...] + eps * xhi_ref[...]], axis=1)
    h = jnp.dot(a, w_ref[...], preferred_element_type=jnp.float32,
                precision=lax.Precision.DEFAULT) + b_ref[...]
    h_ref[...] = h

    @pl.when(pl.program_id(0) == 0)
    def _():
        s_ref[...] = jnp.zeros_like(s_ref)
        ss_ref[...] = jnp.zeros_like(ss_ref)

    s_ref[...] += jnp.sum(h, axis=0, keepdims=True)
    ss_ref[...] += jnp.sum(h * h, axis=0, keepdims=True)


def _m1(agg_lo, agg_hi, x_lo, x_hi, eps, w1, b1):
    return pl.pallas_call(
        _m1_body,
        grid=(GRID,),
        in_specs=[
            pl.BlockSpec((ROW_BLK, HALF), lambda i: (i, 0)),
            pl.BlockSpec((ROW_BLK, HALF), lambda i: (i, 0)),
            pl.BlockSpec((ROW_BLK, HALF), lambda i: (i, 0)),
            pl.BlockSpec((ROW_BLK, HALF), lambda i: (i, 0)),
            pl.BlockSpec(memory_space=pltpu.SMEM),
            pl.BlockSpec((D, D), lambda i: (0, 0)),
            pl.BlockSpec((1, D), lambda i: (0, 0)),
        ],
        out_specs=(
            pl.BlockSpec((ROW_BLK, D), lambda i: (i, 0)),
            pl.BlockSpec((1, D), lambda i: (0, 0)),
            pl.BlockSpec((1, D), lambda i: (0, 0)),
        ),
        out_shape=(
            jax.ShapeDtypeStruct((N, D), jnp.float32),
            jax.ShapeDtypeStruct((1, D), jnp.float32),
            jax.ShapeDtypeStruct((1, D), jnp.float32),
        ),
    )(agg_lo, agg_hi, x_lo, x_hi, eps.reshape(1), w1, b1.reshape(1, D))


def _m2_body(h1_ref, s_ref, ss_ref, g_ref, be_ref, w_ref, b_ref,
             h2_ref, s2_ref, ss2_ref):
    m = s_ref[...] / N
    v = ss_ref[...] / N - m * m
    scale = g_ref[...] / jnp.sqrt(v + BN_EPS)
    shift = be_ref[...] - m * scale
    hn = jnp.maximum(h1_ref[...] * scale + shift, 0.0)
    h2 = jnp.dot(hn, w_ref[...], preferred_element_type=jnp.float32,
                 precision=lax.Precision.DEFAULT) + b_ref[...]
    h2_ref[...] = h2

    @pl.when(pl.program_id(0) == 0)
    def _():
        s2_ref[...] = jnp.zeros_like(s2_ref)
        ss2_ref[...] = jnp.zeros_like(ss2_ref)

    s2_ref[...] += jnp.sum(h2, axis=0, keepdims=True)
    ss2_ref[...] += jnp.sum(h2 * h2, axis=0, keepdims=True)


def _m2(h1, s1, ss1, g1, be1, w2, b2):
    vec = pl.BlockSpec((1, D), lambda i: (0, 0))
    return pl.pallas_call(
        _m2_body,
        grid=(GRID,),
        in_specs=[
            pl.BlockSpec((ROW_BLK, D), lambda i: (i, 0)),
            vec, vec, vec, vec,
            pl.BlockSpec((D, D), lambda i: (0, 0)),
            vec,
        ],
        out_specs=(
            pl.BlockSpec((ROW_BLK, D), lambda i: (i, 0)),
            pl.BlockSpec((1, D), lambda i: (0, 0)),
            pl.BlockSpec((1, D), lambda i: (0, 0)),
        ),
        out_shape=(
            jax.ShapeDtypeStruct((N, D), jnp.float32),
            jax.ShapeDtypeStruct((1, D), jnp.float32),
            jax.ShapeDtypeStruct((1, D), jnp.float32),
        ),
    )(h1, s1, ss1, g1.reshape(1, D), be1.reshape(1, D), w2, b2.reshape(1, D))


def _f0_body(h_ref, s_ref, ss_ref, g_ref, b_ref, olo_ref, ohi_ref):
    m = s_ref[...] / N
    v = ss_ref[...] / N - m * m
    scale = g_ref[...] / jnp.sqrt(v + BN_EPS)
    shift = b_ref[...] - m * scale
    o = jnp.maximum(h_ref[...] * scale + shift, 0.0)
    olo_ref[...] = o[:, :HALF]
    ohi_ref[...] = o[:, HALF:]


def _f0(h2, s2, ss2, go, bo):
    vec = pl.BlockSpec((1, D), lambda i: (0, 0))
    return pl.pallas_call(
        _f0_body,
        grid=(GRID,),
        in_specs=[pl.BlockSpec((ROW_BLK, D), lambda i: (i, 0)),
                  vec, vec, vec, vec],
        out_specs=tuple(pl.BlockSpec((ROW_BLK, HALF), lambda i: (i, 0))
                        for _ in range(2)),
        out_shape=tuple(jax.ShapeDtypeStruct((N, HALF), jnp.float32)
                        for _ in range(2)),
    )(h2, s2, ss2, go.reshape(1, D), bo.reshape(1, D))


def _f1_body(h_ref, s_ref, ss_ref, g_ref, b_ref, o_ref):
    m = s_ref[...] / N
    v = ss_ref[...] / N - m * m
    scale = g_ref[...] / jnp.sqrt(v + BN_EPS)
    shift = b_ref[...] - m * scale
    o_ref[...] = h_ref[...] * scale + shift


def _f1(h2, s2, ss2, go, bo):
    vec = pl.BlockSpec((1, D), lambda i: (0, 0))
    return pl.pallas_call(
        _f1_body,
        grid=(GRID,),
        in_specs=[pl.BlockSpec((ROW_BLK, D), lambda i: (i, 0)),
                  vec, vec, vec, vec],
        out_specs=pl.BlockSpec((ROW_BLK, D), lambda i: (i, 0)),
        out_shape=jax.ShapeDtypeStruct((N, D), jnp.float32),
    )(h2, s2, ss2, go.reshape(1, D), bo.reshape(1, D))


def kernel(x, edge_index, eps0, W1_0, b1_0, g1_0, be1_0, W2_0, b2_0, go_0,
           bo_0, eps1, W1_1, b1_1, g1_1, be1_1, W2_1, b2_1, go_1, bo_1):
    src = edge_index[0]
    dst = edge_index[1]
    src_pad = jnp.zeros((NS, PAD_PER_TILE), jnp.int32)
    dst_pad = (N + DUMMY_PER_TILE * jnp.arange(NS, dtype=jnp.int32)[:, None]
               + jnp.arange(PAD_PER_TILE, dtype=jnp.int32)[None, :]
               % DUMMY_PER_TILE)
    src_r = jnp.concatenate(
        [src.reshape(NS, E // NS), src_pad], axis=1
    ).reshape(NS, CHUNKS_STORED, CHUNK)
    dst_r = jnp.concatenate(
        [dst.reshape(NS, E // NS), dst_pad], axis=1
    ).reshape(NS, CHUNKS_STORED, CHUNK)

    x_lo, x_hi, rx_lo, rx_hi = _prep(x)
    agg_lo, agg_hi = _sc_agg(rx_lo, rx_hi, x_lo, x_hi, src_r, dst_r)
    h1, s1, ss1 = _m1(agg_lo, agg_hi, x_lo, x_hi, eps0, W1_0, b1_0)
    h2, s2, ss2 = _m2(h1, s1, ss1, g1_0, be1_0, W2_0, b2_0)
    o_lo, o_hi = _f0(h2, s2, ss2, go_0, bo_0)

    agg_lo, agg_hi = _sc_agg(o_lo, o_hi, o_lo, o_hi, src_r, dst_r)
    h1, s1, ss1 = _m1(agg_lo, agg_hi, o_lo, o_hi, eps1, W1_1, b1_1)
    h2, s2, ss2 = _m2(h1, s1, ss1, g1_1, be1_1, W2_1, b2_1)
    return _f1(h2, s2, ss2, go_1, bo_1)


# speedup vs baseline: 1.0004x; 1.0004x over previous
import jax
import jax.numpy as jnp
from jax import lax
from jax.experimental import pallas as pl
from jax.experimental.pallas import tpu as pltpu
from jax.experimental.pallas import tpu_sc as plsc

N = 10000
E = 160000
D = 256
HALF = 128
NS = 16
CHUNK = 128
NBUF = 2
CHUNKS_STORED = 80
CHUNKS_PER_TILE = 79
EPT = CHUNKS_STORED * CHUNK
PAD_PER_TILE = EPT - E // NS
DUMMY_PER_TILE = 8
N_PAD = N + NS * DUMMY_PER_TILE
ROWS_PER_TILE = 624
TAIL_ROWS = N - NS * ROWS_PER_TILE
BN_EPS = 1e-5


def _sc_agg_body(t_lo, t_hi, i_lo, i_hi, src_r, dst_r,
                 out_lo, out_hi, acc, src_v, dst_v,
                 buf0, buf1, sg0, sg1, ss0, ss1):
    c = lax.axis_index("c")
    s = lax.axis_index("s")
    r0 = pl.multiple_of(s * ROWS_PER_TILE, 8)
    tail0 = NS * ROWS_PER_TILE

    def run_half(tab, ini, out):
        pltpu.sync_copy(ini.at[pl.ds(r0, ROWS_PER_TILE)],
                        acc.at[pl.ds(r0, ROWS_PER_TILE)])

        @pl.when(s == NS - 1)
        def _():
            pltpu.sync_copy(ini.at[pl.ds(tail0, TAIL_ROWS)],
                            acc.at[pl.ds(tail0, TAIL_ROWS)])

        plsc.subcore_barrier()

        def pair(t0, t1):
            d0 = pltpu.async_copy(tab.at[src_v.at[t0]], buf0, sg0)
            d1 = pltpu.async_copy(tab.at[src_v.at[t1]], buf1, sg1)
            d0.wait()
            s0 = pltpu.async_copy(buf0, acc.at[dst_v.at[t0]], ss0,
                                  add=True)
            d1.wait()
            s1 = pltpu.async_copy(buf1, acc.at[dst_v.at[t1]], ss1,
                                  add=True)
            s0.wait()
            s1.wait()

        for p0, pn, do in ((0, 64, 64), (64, 16, 15)):
            pltpu.sync_copy(src_r.at[s, pl.ds(p0, pn)],
                            src_v.at[pl.ds(0, pn)])
            pltpu.sync_copy(dst_r.at[s, pl.ds(p0, pn)],
                            dst_v.at[pl.ds(0, pn)])

            def body(g, carry):
                pair(g * 2, g * 2 + 1)
                return carry

            lax.fori_loop(0, do // 2, body, 0)
            if do % 2:
                tl = do - 1
                dl = pltpu.async_copy(tab.at[src_v.at[tl]], buf0, sg0)
                dl.wait()
                sl = pltpu.async_copy(buf0, acc.at[dst_v.at[tl]], ss0,
                                      add=True)
                sl.wait()
        plsc.subcore_barrier()
        pltpu.sync_copy(acc.at[pl.ds(r0, ROWS_PER_TILE)],
                        out.at[pl.ds(r0, ROWS_PER_TILE)])

        @pl.when(s == NS - 1)
        def _():
            pltpu.sync_copy(acc.at[pl.ds(tail0, TAIL_ROWS)],
                            out.at[pl.ds(tail0, TAIL_ROWS)])

    pl.when(c == 0)(lambda: run_half(t_lo, i_lo, out_lo))
    pl.when(c == 1)(lambda: run_half(t_hi, i_hi, out_hi))


def _sc_agg(table_lo, table_hi, init_lo, init_hi, src_r, dst_r):
    mesh = plsc.VectorSubcoreMesh(core_axis_name="c", subcore_axis_name="s")
    f = pl.kernel(
        _sc_agg_body,
        out_type=(
            jax.ShapeDtypeStruct((N, HALF), jnp.float32),
            jax.ShapeDtypeStruct((N, HALF), jnp.float32),
        ),
        mesh=mesh,
        scratch_types=(
            [
                pltpu.VMEM_SHARED((N_PAD, HALF), jnp.float32),
                pltpu.VMEM((64, CHUNK), jnp.int32),
                pltpu.VMEM((64, CHUNK), jnp.int32),
            ]
            + [pltpu.VMEM((CHUNK, HALF), jnp.float32)] * NBUF
            + [pltpu.SemaphoreType.DMA] * (2 * NBUF)
        ),
    )
    return f(table_lo, table_hi, init_lo, init_hi, src_r, dst_r)


ROW_BLK = 5000
GRID = N // ROW_BLK


def _prep_body(x_ref, xlo_ref, xhi_ref, rlo_ref, rhi_ref):
    x = x_ref[...]
    xlo_ref[...] = x[:, :HALF]
    xhi_ref[...] = x[:, HALF:]
    r = jnp.maximum(x, 0.0)
    rlo_ref[...] = r[:, :HALF]
    rhi_ref[...] = r[:, HALF:]


def _prep(x):
    return pl.pallas_call(
        _prep_body,
        grid=(GRID,),
        in_specs=[pl.BlockSpec((ROW_BLK, D), lambda i: (i, 0))],
        out_specs=tuple(pl.BlockSpec((ROW_BLK, HALF), lambda i: (i, 0))
                        for _ in range(4)),
        out_shape=tuple(jax.ShapeDtypeStruct((N, HALF), jnp.float32)
                        for _ in range(4)),
    )(x)


def _m1_body(alo_ref, ahi_ref, xlo_ref, xhi_ref, eps_ref, w_ref, b_ref,
             h_ref, s_ref, ss_ref):
    eps = eps_ref[0]
    a = jnp.concatenate(
        [alo_ref[...] + eps * xlo_ref[...],
         ahi_ref[...] + eps * xhi_ref[...]], axis=1)
    h = jnp.dot(a, w_ref[...], preferred_element_type=jnp.float32) + b_ref[...]
    h_ref[...] = h

    @pl.when(pl.program_id(0) == 0)
    def _():
        s_ref[...] = jnp.zeros_like(s_ref)
        ss_ref[...] = jnp.zeros_like(ss_ref)

    s_ref[...] += jnp.sum(h, axis=0, keepdims=True)
    ss_ref[...] += jnp.sum(h * h, axis=0, keepdims=True)


def _m1(agg_lo, agg_hi, x_lo, x_hi, eps, w1, b1):
    return pl.pallas_call(
        _m1_body,
        grid=(GRID,),
        in_specs=[
            pl.BlockSpec((ROW_BLK, HALF), lambda i: (i, 0)),
            pl.BlockSpec((ROW_BLK, HALF), lambda i: (i, 0)),
            pl.BlockSpec((ROW_BLK, HALF), lambda i: (i, 0)),
            pl.BlockSpec((ROW_BLK, HALF), lambda i: (i, 0)),
            pl.BlockSpec(memory_space=pltpu.SMEM),
            pl.BlockSpec((D, D), lambda i: (0, 0)),
            pl.BlockSpec((1, D), lambda i: (0, 0)),
        ],
        out_specs=(
            pl.BlockSpec((ROW_BLK, D), lambda i: (i, 0)),
            pl.BlockSpec((1, D), lambda i: (0, 0)),
            pl.BlockSpec((1, D), lambda i: (0, 0)),
        ),
        out_shape=(
            jax.ShapeDtypeStruct((N, D), jnp.float32),
            jax.ShapeDtypeStruct((1, D), jnp.float32),
            jax.ShapeDtypeStruct((1, D), jnp.float32),
        ),
    )(agg_lo, agg_hi, x_lo, x_hi, eps.reshape(1), w1, b1.reshape(1, D))


def _m2_body(h1_ref, s_ref, ss_ref, g_ref, be_ref, w_ref, b_ref,
             h2_ref, s2_ref, ss2_ref):
    m = s_ref[...] / N
    v = ss_ref[...] / N - m * m
    scale = g_ref[...] / jnp.sqrt(v + BN_EPS)
    shift = be_ref[...] - m * scale
    hn = jnp.maximum(h1_ref[...] * scale + shift, 0.0)
    h2 = jnp.dot(hn, w_ref[...], preferred_element_type=jnp.float32) + b_ref[...]
    h2_ref[...] = h2

    @pl.when(pl.program_id(0) == 0)
    def _():
        s2_ref[...] = jnp.zeros_like(s2_ref)
        ss2_ref[...] = jnp.zeros_like(ss2_ref)

    s2_ref[...] += jnp.sum(h2, axis=0, keepdims=True)
    ss2_ref[...] += jnp.sum(h2 * h2, axis=0, keepdims=True)


def _m2(h1, s1, ss1, g1, be1, w2, b2):
    vec = pl.BlockSpec((1, D), lambda i: (0, 0))
    return pl.pallas_call(
        _m2_body,
        grid=(GRID,),
        in_specs=[
            pl.BlockSpec((ROW_BLK, D), lambda i: (i, 0)),
            vec, vec, vec, vec,
            pl.BlockSpec((D, D), lambda i: (0, 0)),
            vec,
        ],
        out_specs=(
            pl.BlockSpec((ROW_BLK, D), lambda i: (i, 0)),
            pl.BlockSpec((1, D), lambda i: (0, 0)),
            pl.BlockSpec((1, D), lambda i: (0, 0)),
        ),
        out_shape=(
            jax.ShapeDtypeStruct((N, D), jnp.float32),
            jax.ShapeDtypeStruct((1, D), jnp.float32),
            jax.ShapeDtypeStruct((1, D), jnp.float32),
        ),
    )(h1, s1, ss1, g1.reshape(1, D), be1.reshape(1, D), w2, b2.reshape(1, D))


def _f0_body(h_ref, s_ref, ss_ref, g_ref, b_ref, olo_ref, ohi_ref):
    m = s_ref[...] / N
    v = ss_ref[...] / N - m * m
    scale = g_ref[...] / jnp.sqrt(v + BN_EPS)
    shift = b_ref[...] - m * scale
    o = jnp.maximum(h_ref[...] * scale + shift, 0.0)
    olo_ref[...] = o[:, :HALF]
    ohi_ref[...] = o[:, HALF:]


def _f0(h2, s2, ss2, go, bo):
    vec = pl.BlockSpec((1, D), lambda i: (0, 0))
    return pl.pallas_call(
        _f0_body,
        grid=(GRID,),
        in_specs=[pl.BlockSpec((ROW_BLK, D), lambda i: (i, 0)),
                  vec, vec, vec, vec],
        out_specs=tuple(pl.BlockSpec((ROW_BLK, HALF), lambda i: (i, 0))
                        for _ in range(2)),
        out_shape=tuple(jax.ShapeDtypeStruct((N, HALF), jnp.float32)
                        for _ in range(2)),
    )(h2, s2, ss2, go.reshape(1, D), bo.reshape(1, D))


def _f1_body(h_ref, s_ref, ss_ref, g_ref, b_ref, o_ref):
    m = s_ref[...] / N
    v = ss_ref[...] / N - m * m
    scale = g_ref[...] / jnp.sqrt(v + BN_EPS)
    shift = b_ref[...] - m * scale
    o_ref[...] = h_ref[...] * scale + shift


def _f1(h2, s2, ss2, go, bo):
    vec = pl.BlockSpec((1, D), lambda i: (0, 0))
    return pl.pallas_call(
        _f1_body,
        grid=(GRID,),
        in_specs=[pl.BlockSpec((ROW_BLK, D), lambda i: (i, 0)),
                  vec, vec, vec, vec],
        out_specs=pl.BlockSpec((ROW_BLK, D), lambda i: (i, 0)),
        out_shape=jax.ShapeDtypeStruct((N, D), jnp.float32),
    )(h2, s2, ss2, go.reshape(1, D), bo.reshape(1, D))


def kernel(x, edge_index, eps0, W1_0, b1_0, g1_0, be1_0, W2_0, b2_0, go_0,
           bo_0, eps1, W1_1, b1_1, g1_1, be1_1, W2_1, b2_1, go_1, bo_1):
    src = edge_index[0]
    dst = edge_index[1]
    src_pad = jnp.zeros((NS, PAD_PER_TILE), jnp.int32)
    dst_pad = (N + DUMMY_PER_TILE * jnp.arange(NS, dtype=jnp.int32)[:, None]
               + jnp.arange(PAD_PER_TILE, dtype=jnp.int32)[None, :]
               % DUMMY_PER_TILE)
    src_r = jnp.concatenate(
        [src.reshape(NS, E // NS), src_pad], axis=1
    ).reshape(NS, CHUNKS_STORED, CHUNK)
    dst_r = jnp.concatenate(
        [dst.reshape(NS, E // NS), dst_pad], axis=1
    ).reshape(NS, CHUNKS_STORED, CHUNK)

    x_lo, x_hi, rx_lo, rx_hi = _prep(x)
    agg_lo, agg_hi = _sc_agg(rx_lo, rx_hi, x_lo, x_hi, src_r, dst_r)
    h1, s1, ss1 = _m1(agg_lo, agg_hi, x_lo, x_hi, eps0, W1_0, b1_0)
    h2, s2, ss2 = _m2(h1, s1, ss1, g1_0, be1_0, W2_0, b2_0)
    o_lo, o_hi = _f0(h2, s2, ss2, go_0, bo_0)

    agg_lo, agg_hi = _sc_agg(o_lo, o_hi, o_lo, o_hi, src_r, dst_r)
    h1, s1, ss1 = _m1(agg_lo, agg_hi, o_lo, o_hi, eps1, W1_1, b1_1)
    h2, s2, ss2 = _m2(h1, s1, ss1, g1_1, be1_1, W2_1, b2_1)
    return _f1(h2, s2, ss2, go_1, bo_1)


# speedup vs baseline: 1.0538x; 1.0533x over previous
import jax
import jax.numpy as jnp
from jax import lax
from jax.experimental import pallas as pl
from jax.experimental.pallas import tpu as pltpu
from jax.experimental.pallas import tpu_sc as plsc

N = 10000
E = 160000
D = 256
HALF = 128
NS = 16
CHUNK = 128
NBUF = 2
CHUNKS_STORED = 80
CHUNKS_PER_TILE = 79
EPT = CHUNKS_STORED * CHUNK
PAD_PER_TILE = EPT - E // NS
DUMMY_PER_TILE = 8
N_PAD = N + NS * DUMMY_PER_TILE
ROWS_PER_TILE = 624
TAIL_ROWS = N - NS * ROWS_PER_TILE
BN_EPS = 1e-5


def _sc_agg_body(t_lo, t_hi, i_lo, i_hi, src_r, dst_r,
                 out_lo, out_hi, acc, src_v, dst_v,
                 buf0, buf1, sg0, sg1, ss0, ss1):
    c = lax.axis_index("c")
    s = lax.axis_index("s")
    r0 = pl.multiple_of(s * ROWS_PER_TILE, 8)
    tail0 = NS * ROWS_PER_TILE

    def run_half(tab, ini, out):
        pltpu.sync_copy(ini.at[pl.ds(r0, ROWS_PER_TILE)],
                        acc.at[pl.ds(r0, ROWS_PER_TILE)])

        @pl.when(s == NS - 1)
        def _():
            pltpu.sync_copy(ini.at[pl.ds(tail0, TAIL_ROWS)],
                            acc.at[pl.ds(tail0, TAIL_ROWS)])

        plsc.subcore_barrier()

        bufs = (buf0, buf1)
        sem_g = (sg0, sg1)
        sem_s = (ss0, ss1)

        def gath(t, b):
            return pltpu.async_copy(tab.at[src_v.at[t]], bufs[b],
                                    sem_g[b])

        def scat(t, b):
            return pltpu.async_copy(bufs[b], acc.at[dst_v.at[t]],
                                    sem_s[b], add=True)

        def chain(ts):
            k = len(ts)
            d = [None] * k
            sd = [None] * k
            d[0] = gath(ts[0], 0)
            if k > 1:
                d[1] = gath(ts[1], 1)
            for i in range(k):
                b = i & 1
                d[i].wait()
                sd[i] = scat(ts[i], b)
                if i >= 1 and i + 1 < k:
                    sd[i - 1].wait()
                    d[i + 1] = gath(ts[i + 1], 1 - b)
            if k > 1:
                sd[k - 2].wait()
            sd[k - 1].wait()

        for p0, pn, do in ((0, 64, 64), (64, 16, 15)):
            pltpu.sync_copy(src_r.at[s, pl.ds(p0, pn)],
                            src_v.at[pl.ds(0, pn)])
            pltpu.sync_copy(dst_r.at[s, pl.ds(p0, pn)],
                            dst_v.at[pl.ds(0, pn)])
            if do % 16 == 0:
                def body(g, carry):
                    chain([g * 16 + i for i in range(16)])
                    return carry

                lax.fori_loop(0, do // 16, body, 0)
            else:
                chain(list(range(do)))
        plsc.subcore_barrier()
        pltpu.sync_copy(acc.at[pl.ds(r0, ROWS_PER_TILE)],
                        out.at[pl.ds(r0, ROWS_PER_TILE)])

        @pl.when(s == NS - 1)
        def _():
            pltpu.sync_copy(acc.at[pl.ds(tail0, TAIL_ROWS)],
                            out.at[pl.ds(tail0, TAIL_ROWS)])

    pl.when(c == 0)(lambda: run_half(t_lo, i_lo, out_lo))
    pl.when(c == 1)(lambda: run_half(t_hi, i_hi, out_hi))


def _sc_agg(table_lo, table_hi, init_lo, init_hi, src_r, dst_r):
    mesh = plsc.VectorSubcoreMesh(core_axis_name="c", subcore_axis_name="s")
    f = pl.kernel(
        _sc_agg_body,
        out_type=(
            jax.ShapeDtypeStruct((N, HALF), jnp.float32),
            jax.ShapeDtypeStruct((N, HALF), jnp.float32),
        ),
        mesh=mesh,
        scratch_types=(
            [
                pltpu.VMEM_SHARED((N_PAD, HALF), jnp.float32),
                pltpu.VMEM((64, CHUNK), jnp.int32),
                pltpu.VMEM((64, CHUNK), jnp.int32),
            ]
            + [pltpu.VMEM((CHUNK, HALF), jnp.float32)] * NBUF
            + [pltpu.SemaphoreType.DMA] * (2 * NBUF)
        ),
    )
    return f(table_lo, table_hi, init_lo, init_hi, src_r, dst_r)


ROW_BLK = 5000
GRID = N // ROW_BLK


def _prep_body(x_ref, xlo_ref, xhi_ref, rlo_ref, rhi_ref):
    x = x_ref[...]
    xlo_ref[...] = x[:, :HALF]
    xhi_ref[...] = x[:, HALF:]
    r = jnp.maximum(x, 0.0)
    rlo_ref[...] = r[:, :HALF]
    rhi_ref[...] = r[:, HALF:]


def _prep(x):
    return pl.pallas_call(
        _prep_body,
        grid=(GRID,),
        in_specs=[pl.BlockSpec((ROW_BLK, D), lambda i: (i, 0))],
        out_specs=tuple(pl.BlockSpec((ROW_BLK, HALF), lambda i: (i, 0))
                        for _ in range(4)),
        out_shape=tuple(jax.ShapeDtypeStruct((N, HALF), jnp.float32)
                        for _ in range(4)),
    )(x)


def _m1_body(alo_ref, ahi_ref, xlo_ref, xhi_ref, eps_ref, w_ref, b_ref,
             h_ref, s_ref, ss_ref):
    eps = eps_ref[0]
    a = jnp.concatenate(
        [alo_ref[...] + eps * xlo_ref[...],
         ahi_ref[...] + eps * xhi_ref[...]], axis=1)
    h = jnp.dot(a, w_ref[...], preferred_element_type=jnp.float32) + b_ref[...]
    h_ref[...] = h

    @pl.when(pl.program_id(0) == 0)
    def _():
        s_ref[...] = jnp.zeros_like(s_ref)
        ss_ref[...] = jnp.zeros_like(ss_ref)

    s_ref[...] += jnp.sum(h, axis=0, keepdims=True)
    ss_ref[...] += jnp.sum(h * h, axis=0, keepdims=True)


def _m1(agg_lo, agg_hi, x_lo, x_hi, eps, w1, b1):
    return pl.pallas_call(
        _m1_body,
        grid=(GRID,),
        in_specs=[
            pl.BlockSpec((ROW_BLK, HALF), lambda i: (i, 0)),
            pl.BlockSpec((ROW_BLK, HALF), lambda i: (i, 0)),
            pl.BlockSpec((ROW_BLK, HALF), lambda i: (i, 0)),
            pl.BlockSpec((ROW_BLK, HALF), lambda i: (i, 0)),
            pl.BlockSpec(memory_space=pltpu.SMEM),
            pl.BlockSpec((D, D), lambda i: (0, 0)),
            pl.BlockSpec((1, D), lambda i: (0, 0)),
        ],
        out_specs=(
            pl.BlockSpec((ROW_BLK, D), lambda i: (i, 0)),
            pl.BlockSpec((1, D), lambda i: (0, 0)),
            pl.BlockSpec((1, D), lambda i: (0, 0)),
        ),
        out_shape=(
            jax.ShapeDtypeStruct((N, D), jnp.float32),
            jax.ShapeDtypeStruct((1, D), jnp.float32),
            jax.ShapeDtypeStruct((1, D), jnp.float32),
        ),
    )(agg_lo, agg_hi, x_lo, x_hi, eps.reshape(1), w1, b1.reshape(1, D))


def _m2_body(h1_ref, s_ref, ss_ref, g_ref, be_ref, w_ref, b_ref,
             h2_ref, s2_ref, ss2_ref):
    m = s_ref[...] / N
    v = ss_ref[...] / N - m * m
    scale = g_ref[...] / jnp.sqrt(v + BN_EPS)
    shift = be_ref[...] - m * scale
    hn = jnp.maximum(h1_ref[...] * scale + shift, 0.0)
    h2 = jnp.dot(hn, w_ref[...], preferred_element_type=jnp.float32) + b_ref[...]
    h2_ref[...] = h2

    @pl.when(pl.program_id(0) == 0)
    def _():
        s2_ref[...] = jnp.zeros_like(s2_ref)
        ss2_ref[...] = jnp.zeros_like(ss2_ref)

    s2_ref[...] += jnp.sum(h2, axis=0, keepdims=True)
    ss2_ref[...] += jnp.sum(h2 * h2, axis=0, keepdims=True)


def _m2(h1, s1, ss1, g1, be1, w2, b2):
    vec = pl.BlockSpec((1, D), lambda i: (0, 0))
    return pl.pallas_call(
        _m2_body,
        grid=(GRID,),
        in_specs=[
            pl.BlockSpec((ROW_BLK, D), lambda i: (i, 0)),
            vec, vec, vec, vec,
            pl.BlockSpec((D, D), lambda i: (0, 0)),
            vec,
        ],
        out_specs=(
            pl.BlockSpec((ROW_BLK, D), lambda i: (i, 0)),
            pl.BlockSpec((1, D), lambda i: (0, 0)),
            pl.BlockSpec((1, D), lambda i: (0, 0)),
        ),
        out_shape=(
            jax.ShapeDtypeStruct((N, D), jnp.float32),
            jax.ShapeDtypeStruct((1, D), jnp.float32),
            jax.ShapeDtypeStruct((1, D), jnp.float32),
        ),
    )(h1, s1, ss1, g1.reshape(1, D), be1.reshape(1, D), w2, b2.reshape(1, D))


def _f0_body(h_ref, s_ref, ss_ref, g_ref, b_ref, olo_ref, ohi_ref):
    m = s_ref[...] / N
    v = ss_ref[...] / N - m * m
    scale = g_ref[...] / jnp.sqrt(v + BN_EPS)
    shift = b_ref[...] - m * scale
    o = jnp.maximum(h_ref[...] * scale + shift, 0.0)
    olo_ref[...] = o[:, :HALF]
    ohi_ref[...] = o[:, HALF:]


def _f0(h2, s2, ss2, go, bo):
    vec = pl.BlockSpec((1, D), lambda i: (0, 0))
    return pl.pallas_call(
        _f0_body,
        grid=(GRID,),
        in_specs=[pl.BlockSpec((ROW_BLK, D), lambda i: (i, 0)),
                  vec, vec, vec, vec],
        out_specs=tuple(pl.BlockSpec((ROW_BLK, HALF), lambda i: (i, 0))
                        for _ in range(2)),
        out_shape=tuple(jax.ShapeDtypeStruct((N, HALF), jnp.float32)
                        for _ in range(2)),
    )(h2, s2, ss2, go.reshape(1, D), bo.reshape(1, D))


def _f1_body(h_ref, s_ref, ss_ref, g_ref, b_ref, o_ref):
    m = s_ref[...] / N
    v = ss_ref[...] / N - m * m
    scale = g_ref[...] / jnp.sqrt(v + BN_EPS)
    shift = b_ref[...] - m * scale
    o_ref[...] = h_ref[...] * scale + shift


def _f1(h2, s2, ss2, go, bo):
    vec = pl.BlockSpec((1, D), lambda i: (0, 0))
    return pl.pallas_call(
        _f1_body,
        grid=(GRID,),
        in_specs=[pl.BlockSpec((ROW_BLK, D), lambda i: (i, 0)),
                  vec, vec, vec, vec],
        out_specs=pl.BlockSpec((ROW_BLK, D), lambda i: (i, 0)),
        out_shape=jax.ShapeDtypeStruct((N, D), jnp.float32),
    )(h2, s2, ss2, go.reshape(1, D), bo.reshape(1, D))


def kernel(x, edge_index, eps0, W1_0, b1_0, g1_0, be1_0, W2_0, b2_0, go_0,
           bo_0, eps1, W1_1, b1_1, g1_1, be1_1, W2_1, b2_1, go_1, bo_1):
    src = edge_index[0]
    dst = edge_index[1]
    src_pad = jnp.zeros((NS, PAD_PER_TILE), jnp.int32)
    dst_pad = (N + DUMMY_PER_TILE * jnp.arange(NS, dtype=jnp.int32)[:, None]
               + jnp.arange(PAD_PER_TILE, dtype=jnp.int32)[None, :]
               % DUMMY_PER_TILE)
    src_r = jnp.concatenate(
        [src.reshape(NS, E // NS), src_pad], axis=1
    ).reshape(NS, CHUNKS_STORED, CHUNK)
    dst_r = jnp.concatenate(
        [dst.reshape(NS, E // NS), dst_pad], axis=1
    ).reshape(NS, CHUNKS_STORED, CHUNK)

    x_lo, x_hi, rx_lo, rx_hi = _prep(x)
    agg_lo, agg_hi = _sc_agg(rx_lo, rx_hi, x_lo, x_hi, src_r, dst_r)
    h1, s1, ss1 = _m1(agg_lo, agg_hi, x_lo, x_hi, eps0, W1_0, b1_0)
    h2, s2, ss2 = _m2(h1, s1, ss1, g1_0, be1_0, W2_0, b2_0)
    o_lo, o_hi = _f0(h2, s2, ss2, go_0, bo_0)

    agg_lo, agg_hi = _sc_agg(o_lo, o_hi, o_lo, o_hi, src_r, dst_r)
    h1, s1, ss1 = _m1(agg_lo, agg_hi, o_lo, o_hi, eps1, W1_1, b1_1)
    h2, s2, ss2 = _m2(h1, s1, ss1, g1_1, be1_1, W2_1, b2_1)
    return _f1(h2, s2, ss2, go_1, bo_1)


# speedup vs baseline: 1.0603x; 1.0062x over previous
import jax
import jax.numpy as jnp
from jax import lax
from jax.experimental import pallas as pl
from jax.experimental.pallas import tpu as pltpu
from jax.experimental.pallas import tpu_sc as plsc

N = 10000
E = 160000
D = 256
HALF = 128
NS = 16
CHUNK = 128
NBUF = 2
CHUNKS_STORED = 80
CHUNKS_PER_TILE = 79
EPT = CHUNKS_STORED * CHUNK
PAD_PER_TILE = EPT - E // NS
DUMMY_PER_TILE = 8
N_PAD = N + NS * DUMMY_PER_TILE
ROWS_PER_TILE = 624
TAIL_ROWS = N - NS * ROWS_PER_TILE
BN_EPS = 1e-5


def _sc_agg_body(t_lo, t_hi, i_lo, i_hi, src_r, dst_r,
                 out_lo, out_hi, acc, src_v, dst_v,
                 buf0, buf1, sg0, sg1, ss0, ss1):
    c = lax.axis_index("c")
    s = lax.axis_index("s")
    r0 = pl.multiple_of(s * ROWS_PER_TILE, 8)
    tail0 = NS * ROWS_PER_TILE

    def run_half(tab, ini, out):
        pltpu.sync_copy(ini.at[pl.ds(r0, ROWS_PER_TILE)],
                        acc.at[pl.ds(r0, ROWS_PER_TILE)])

        @pl.when(s == NS - 1)
        def _():
            pltpu.sync_copy(ini.at[pl.ds(tail0, TAIL_ROWS)],
                            acc.at[pl.ds(tail0, TAIL_ROWS)])

        plsc.subcore_barrier()

        bufs = (buf0, buf1)
        sem_g = (sg0, sg1)
        sem_s = (ss0, ss1)

        def gath(t, b):
            return pltpu.async_copy(tab.at[src_v.at[t]], bufs[b],
                                    sem_g[b])

        def scat(t, b):
            return pltpu.async_copy(bufs[b], acc.at[dst_v.at[t]],
                                    sem_s[b], add=True)

        def chain(ts):
            k = len(ts)
            d = [None] * k
            sd = [None] * k
            d[0] = gath(ts[0], 0)
            if k > 1:
                d[1] = gath(ts[1], 1)
            for i in range(k):
                b = i & 1
                d[i].wait()
                sd[i] = scat(ts[i], b)
                if i >= 1 and i + 1 < k:
                    sd[i - 1].wait()
                    d[i + 1] = gath(ts[i + 1], 1 - b)
            if k > 1:
                sd[k - 2].wait()
            sd[k - 1].wait()

        for p0, pn, do in ((0, 64, 64), (64, 16, 15)):
            pltpu.sync_copy(src_r.at[s, pl.ds(p0, pn)],
                            src_v.at[pl.ds(0, pn)])
            pltpu.sync_copy(dst_r.at[s, pl.ds(p0, pn)],
                            dst_v.at[pl.ds(0, pn)])
            if do % 32 == 0:
                def body(g, carry):
                    chain([g * 32 + i for i in range(32)])
                    return carry

                lax.fori_loop(0, do // 32, body, 0)
            else:
                chain(list(range(do)))
        plsc.subcore_barrier()
        pltpu.sync_copy(acc.at[pl.ds(r0, ROWS_PER_TILE)],
                        out.at[pl.ds(r0, ROWS_PER_TILE)])

        @pl.when(s == NS - 1)
        def _():
            pltpu.sync_copy(acc.at[pl.ds(tail0, TAIL_ROWS)],
                            out.at[pl.ds(tail0, TAIL_ROWS)])

    pl.when(c == 0)(lambda: run_half(t_lo, i_lo, out_lo))
    pl.when(c == 1)(lambda: run_half(t_hi, i_hi, out_hi))


def _sc_agg(table_lo, table_hi, init_lo, init_hi, src_r, dst_r):
    mesh = plsc.VectorSubcoreMesh(core_axis_name="c", subcore_axis_name="s")
    f = pl.kernel(
        _sc_agg_body,
        out_type=(
            jax.ShapeDtypeStruct((N, HALF), jnp.float32),
            jax.ShapeDtypeStruct((N, HALF), jnp.float32),
        ),
        mesh=mesh,
        scratch_types=(
            [
                pltpu.VMEM_SHARED((N_PAD, HALF), jnp.float32),
                pltpu.VMEM((64, CHUNK), jnp.int32),
                pltpu.VMEM((64, CHUNK), jnp.int32),
            ]
            + [pltpu.VMEM((CHUNK, HALF), jnp.float32)] * NBUF
            + [pltpu.SemaphoreType.DMA] * (2 * NBUF)
        ),
    )
    return f(table_lo, table_hi, init_lo, init_hi, src_r, dst_r)


ROW_BLK = 5000
GRID = N // ROW_BLK


def _prep_body(x_ref, xlo_ref, xhi_ref, rlo_ref, rhi_ref):
    x = x_ref[...]
    xlo_ref[...] = x[:, :HALF]
    xhi_ref[...] = x[:, HALF:]
    r = jnp.maximum(x, 0.0)
    rlo_ref[...] = r[:, :HALF]
    rhi_ref[...] = r[:, HALF:]


def _prep(x):
    return pl.pallas_call(
        _prep_body,
        grid=(GRID,),
        in_specs=[pl.BlockSpec((ROW_BLK, D), lambda i: (i, 0))],
        out_specs=tuple(pl.BlockSpec((ROW_BLK, HALF), lambda i: (i, 0))
                        for _ in range(4)),
        out_shape=tuple(jax.ShapeDtypeStruct((N, HALF), jnp.float32)
                        for _ in range(4)),
    )(x)


def _m1_body(alo_ref, ahi_ref, xlo_ref, xhi_ref, eps_ref, w_ref, b_ref,
             h_ref, s_ref, ss_ref):
    eps = eps_ref[0]
    a = jnp.concatenate(
        [alo_ref[...] + eps * xlo_ref[...],
         ahi_ref[...] + eps * xhi_ref[...]], axis=1)
    h = jnp.dot(a, w_ref[...], preferred_element_type=jnp.float32) + b_ref[...]
    h_ref[...] = h

    @pl.when(pl.program_id(0) == 0)
    def _():
        s_ref[...] = jnp.zeros_like(s_ref)
        ss_ref[...] = jnp.zeros_like(ss_ref)

    s_ref[...] += jnp.sum(h, axis=0, keepdims=True)
    ss_ref[...] += jnp.sum(h * h, axis=0, keepdims=True)


def _m1(agg_lo, agg_hi, x_lo, x_hi, eps, w1, b1):
    return pl.pallas_call(
        _m1_body,
        grid=(GRID,),
        in_specs=[
            pl.BlockSpec((ROW_BLK, HALF), lambda i: (i, 0)),
            pl.BlockSpec((ROW_BLK, HALF), lambda i: (i, 0)),
            pl.BlockSpec((ROW_BLK, HALF), lambda i: (i, 0)),
            pl.BlockSpec((ROW_BLK, HALF), lambda i: (i, 0)),
            pl.BlockSpec(memory_space=pltpu.SMEM),
            pl.BlockSpec((D, D), lambda i: (0, 0)),
            pl.BlockSpec((1, D), lambda i: (0, 0)),
        ],
        out_specs=(
            pl.BlockSpec((ROW_BLK, D), lambda i: (i, 0)),
            pl.BlockSpec((1, D), lambda i: (0, 0)),
            pl.BlockSpec((1, D), lambda i: (0, 0)),
        ),
        out_shape=(
            jax.ShapeDtypeStruct((N, D), jnp.float32),
            jax.ShapeDtypeStruct((1, D), jnp.float32),
            jax.ShapeDtypeStruct((1, D), jnp.float32),
        ),
    )(agg_lo, agg_hi, x_lo, x_hi, eps.reshape(1), w1, b1.reshape(1, D))


def _m2_body(h1_ref, s_ref, ss_ref, g_ref, be_ref, w_ref, b_ref,
             h2_ref, s2_ref, ss2_ref):
    m = s_ref[...] / N
    v = ss_ref[...] / N - m * m
    scale = g_ref[...] / jnp.sqrt(v + BN_EPS)
    shift = be_ref[...] - m * scale
    hn = jnp.maximum(h1_ref[...] * scale + shift, 0.0)
    h2 = jnp.dot(hn, w_ref[...], preferred_element_type=jnp.float32) + b_ref[...]
    h2_ref[...] = h2

    @pl.when(pl.program_id(0) == 0)
    def _():
        s2_ref[...] = jnp.zeros_like(s2_ref)
        ss2_ref[...] = jnp.zeros_like(ss2_ref)

    s2_ref[...] += jnp.sum(h2, axis=0, keepdims=True)
    ss2_ref[...] += jnp.sum(h2 * h2, axis=0, keepdims=True)


def _m2(h1, s1, ss1, g1, be1, w2, b2):
    vec = pl.BlockSpec((1, D), lambda i: (0, 0))
    return pl.pallas_call(
        _m2_body,
        grid=(GRID,),
        in_specs=[
            pl.BlockSpec((ROW_BLK, D), lambda i: (i, 0)),
            vec, vec, vec, vec,
            pl.BlockSpec((D, D), lambda i: (0, 0)),
            vec,
        ],
        out_specs=(
            pl.BlockSpec((ROW_BLK, D), lambda i: (i, 0)),
            pl.BlockSpec((1, D), lambda i: (0, 0)),
            pl.BlockSpec((1, D), lambda i: (0, 0)),
        ),
        out_shape=(
            jax.ShapeDtypeStruct((N, D), jnp.float32),
            jax.ShapeDtypeStruct((1, D), jnp.float32),
            jax.ShapeDtypeStruct((1, D), jnp.float32),
        ),
    )(h1, s1, ss1, g1.reshape(1, D), be1.reshape(1, D), w2, b2.reshape(1, D))


def _f0_body(h_ref, s_ref, ss_ref, g_ref, b_ref, olo_ref, ohi_ref):
    m = s_ref[...] / N
    v = ss_ref[...] / N - m * m
    scale = g_ref[...] / jnp.sqrt(v + BN_EPS)
    shift = b_ref[...] - m * scale
    o = jnp.maximum(h_ref[...] * scale + shift, 0.0)
    olo_ref[...] = o[:, :HALF]
    ohi_ref[...] = o[:, HALF:]


def _f0(h2, s2, ss2, go, bo):
    vec = pl.BlockSpec((1, D), lambda i: (0, 0))
    return pl.pallas_call(
        _f0_body,
        grid=(GRID,),
        in_specs=[pl.BlockSpec((ROW_BLK, D), lambda i: (i, 0)),
                  vec, vec, vec, vec],
        out_specs=tuple(pl.BlockSpec((ROW_BLK, HALF), lambda i: (i, 0))
                        for _ in range(2)),
        out_shape=tuple(jax.ShapeDtypeStruct((N, HALF), jnp.float32)
                        for _ in range(2)),
    )(h2, s2, ss2, go.reshape(1, D), bo.reshape(1, D))


def _f1_body(h_ref, s_ref, ss_ref, g_ref, b_ref, o_ref):
    m = s_ref[...] / N
    v = ss_ref[...] / N - m * m
    scale = g_ref[...] / jnp.sqrt(v + BN_EPS)
    shift = b_ref[...] - m * scale
    o_ref[...] = h_ref[...] * scale + shift


def _f1(h2, s2, ss2, go, bo):
    vec = pl.BlockSpec((1, D), lambda i: (0, 0))
    return pl.pallas_call(
        _f1_body,
        grid=(GRID,),
        in_specs=[pl.BlockSpec((ROW_BLK, D), lambda i: (i, 0)),
                  vec, vec, vec, vec],
        out_specs=pl.BlockSpec((ROW_BLK, D), lambda i: (i, 0)),
        out_shape=jax.ShapeDtypeStruct((N, D), jnp.float32),
    )(h2, s2, ss2, go.reshape(1, D), bo.reshape(1, D))


def kernel(x, edge_index, eps0, W1_0, b1_0, g1_0, be1_0, W2_0, b2_0, go_0,
           bo_0, eps1, W1_1, b1_1, g1_1, be1_1, W2_1, b2_1, go_1, bo_1):
    src = edge_index[0]
    dst = edge_index[1]
    src_pad = jnp.zeros((NS, PAD_PER_TILE), jnp.int32)
    dst_pad = (N + DUMMY_PER_TILE * jnp.arange(NS, dtype=jnp.int32)[:, None]
               + jnp.arange(PAD_PER_TILE, dtype=jnp.int32)[None, :]
               % DUMMY_PER_TILE)
    src_r = jnp.concatenate(
        [src.reshape(NS, E // NS), src_pad], axis=1
    ).reshape(NS, CHUNKS_STORED, CHUNK)
    dst_r = jnp.concatenate(
        [dst.reshape(NS, E // NS), dst_pad], axis=1
    ).reshape(NS, CHUNKS_STORED, CHUNK)

    x_lo, x_hi, rx_lo, rx_hi = _prep(x)
    agg_lo, agg_hi = _sc_agg(rx_lo, rx_hi, x_lo, x_hi, src_r, dst_r)
    h1, s1, ss1 = _m1(agg_lo, agg_hi, x_lo, x_hi, eps0, W1_0, b1_0)
    h2, s2, ss2 = _m2(h1, s1, ss1, g1_0, be1_0, W2_0, b2_0)
    o_lo, o_hi = _f0(h2, s2, ss2, go_0, bo_0)

    agg_lo, agg_hi = _sc_agg(o_lo, o_hi, o_lo, o_hi, src_r, dst_r)
    h1, s1, ss1 = _m1(agg_lo, agg_hi, o_lo, o_hi, eps1, W1_1, b1_1)
    h2, s2, ss2 = _m2(h1, s1, ss1, g1_1, be1_1, W2_1, b2_1)
    return _f1(h2, s2, ss2, go_1, bo_1)
